# Initial kernel scaffold; baseline (speedup 1.0000x reference)
import functools

import jax
import jax.numpy as jnp
from jax import lax
from jax.experimental import pallas as pl
from jax.experimental.pallas import tpu as pltpu
from jax.experimental.pallas import tpu_sc as plsc

N = 10000
E = 160000
D_IN = 256
H = 512
D_OUT = 256

NC = 2
NS = 16
LANES = 16

EPT = E // NS
CH = 80
NCHUNK = EPT // CH
ROWCH = N // CH

_mesh = plsc.VectorSubcoreMesh(
    core_axis_name="c", subcore_axis_name="s", num_cores=NC, num_subcores=NS
)


def _zero_vmem(ref, rows, width):
    def body(r, _):
        for k in range(width // LANES):
            ref[r, pl.ds(k * LANES, LANES)] = jnp.zeros((LANES,), jnp.float32)
        return 0
    lax.fori_loop(0, rows, body, 0, unroll=False)


def _acc_chunks(s, fn):
    for k in range(ROWCH // NS):
        fn(k * NS + s)
    rem = ROWCH % NS
    if rem:
        @pl.when(s < rem)
        def _():
            fn((ROWCH // NS) * NS + s)


def _deg_body(src3, dst3, dego_hbm, degi_hbm, idx2, ones_v, zb, acc):
    c = lax.axis_index("c")
    s = lax.axis_index("s")

    def fill(r, _):
        ones_v[r, :] = jnp.full((LANES,), 1.0, jnp.float32)
        zb[r, :] = jnp.zeros((LANES,), jnp.float32)
        return 0
    lax.fori_loop(0, CH, fill, 0, unroll=False)

    _acc_chunks(s, lambda ci: pltpu.sync_copy(zb, acc.at[pl.ds(ci * CH, CH)]))
    plsc.subcore_barrier()

    @pl.when(c == 0)
    def _():
        pltpu.sync_copy(src3.at[s], idx2)

    @pl.when(c == 1)
    def _():
        pltpu.sync_copy(dst3.at[s], idx2)

    def eloop(j, _):
        pltpu.sync_copy(ones_v, acc.at[idx2.at[j]], add=True)
        return 0
    lax.fori_loop(0, NCHUNK, eloop, 0, unroll=False)
    plsc.subcore_barrier()

    @pl.when(c == 0)
    def _():
        _acc_chunks(s, lambda ci: pltpu.sync_copy(
            acc.at[pl.ds(ci * CH, CH)], dego_hbm.at[pl.ds(ci * CH, CH)]))

    @pl.when(c == 1)
    def _():
        _acc_chunks(s, lambda ci: pltpu.sync_copy(
            acc.at[pl.ds(ci * CH, CH)], degi_hbm.at[pl.ds(ci * CH, CH)]))


_deg_kernel = functools.partial(
    pl.kernel,
    _deg_body,
    out_type=[
        jax.ShapeDtypeStruct((N, LANES), jnp.float32),
        jax.ShapeDtypeStruct((N, LANES), jnp.float32),
    ],
    mesh=_mesh,
    scratch_types=[
        pltpu.VMEM((NCHUNK, CH), jnp.int32),
        pltpu.VMEM((CH, LANES), jnp.float32),
        pltpu.VMEM((CH, LANES), jnp.float32),
        pltpu.VMEM_SHARED((N, LANES), jnp.float32),
    ],
)()


def _agg_body(src3, dst3, t0, t1, o0, o1, sidx, didx, rows, zb, acc):
    c = lax.axis_index("c")
    s = lax.axis_index("s")

    _zero_vmem(zb, CH, 128)
    _acc_chunks(s, lambda ci: pltpu.sync_copy(zb, acc.at[pl.ds(ci * CH, CH)]))
    plsc.subcore_barrier()

    pltpu.sync_copy(src3.at[s], sidx)
    pltpu.sync_copy(dst3.at[s], didx)

    def make_eloop(tbl):
        def eloop(j, _):
            pltpu.sync_copy(tbl.at[sidx.at[j]], rows)
            pltpu.sync_copy(rows, acc.at[didx.at[j]], add=True)
            return 0
        return eloop

    @pl.when(c == 0)
    def _():
        lax.fori_loop(0, NCHUNK, make_eloop(t0), 0, unroll=False)

    @pl.when(c == 1)
    def _():
        lax.fori_loop(0, NCHUNK, make_eloop(t1), 0, unroll=False)

    plsc.subcore_barrier()

    @pl.when(c == 0)
    def _():
        _acc_chunks(s, lambda ci: pltpu.sync_copy(
            acc.at[pl.ds(ci * CH, CH)], o0.at[pl.ds(ci * CH, CH)]))

    @pl.when(c == 1)
    def _():
        _acc_chunks(s, lambda ci: pltpu.sync_copy(
            acc.at[pl.ds(ci * CH, CH)], o1.at[pl.ds(ci * CH, CH)]))


_agg_kernel = functools.partial(
    pl.kernel,
    _agg_body,
    out_type=[
        jax.ShapeDtypeStruct((N, 128), jnp.float32),
        jax.ShapeDtypeStruct((N, 128), jnp.float32),
    ],
    mesh=_mesh,
    scratch_types=[
        pltpu.VMEM((NCHUNK, CH), jnp.int32),
        pltpu.VMEM((NCHUNK, CH), jnp.int32),
        pltpu.VMEM((CH, 128), jnp.float32),
        pltpu.VMEM((CH, 128), jnp.float32),
        pltpu.VMEM_SHARED((N, 128), jnp.float32),
    ],
)()


BLK = 1000


def _prep_body(x_ref, dego_ref, l_ref, r_ref):
    ns = lax.rsqrt(jnp.maximum(dego_ref[:, 0:1], 1.0))
    xs = x_ref[:, :] * ns
    l_ref[:, :] = xs[:, 0:128]
    r_ref[:, :] = xs[:, 128:256]


def _prep_call(x, dego):
    return pl.pallas_call(
        _prep_body,
        grid=(N // BLK,),
        in_specs=[
            pl.BlockSpec((BLK, D_IN), lambda i: (i, 0)),
            pl.BlockSpec((BLK, LANES), lambda i: (i, 0)),
        ],
        out_specs=[
            pl.BlockSpec((BLK, 128), lambda i: (i, 0)),
            pl.BlockSpec((BLK, 128), lambda i: (i, 0)),
        ],
        out_shape=[
            jax.ShapeDtypeStruct((N, 128), jnp.float32),
            jax.ShapeDtypeStruct((N, 128), jnp.float32),
        ],
    )(x, dego)


def _mid_body(al, ar, dego_ref, degi_ref, w, b, q0, q1, q2, q3):
    m = jnp.dot(al[:, :], w[0:128, :], preferred_element_type=jnp.float32)
    m += jnp.dot(ar[:, :], w[128:256, :], preferred_element_type=jnp.float32)
    nd = lax.rsqrt(jnp.maximum(degi_ref[:, 0:1], 1.0))
    h = jnp.maximum(m * nd + b[:, :], 0.0)
    ns = lax.rsqrt(jnp.maximum(dego_ref[:, 0:1], 1.0))
    hs = h * ns
    q0[:, :] = hs[:, 0:128]
    q1[:, :] = hs[:, 128:256]
    q2[:, :] = hs[:, 256:384]
    q3[:, :] = hs[:, 384:512]


def _mid_call(al, ar, dego, degi, w0, b0):
    return pl.pallas_call(
        _mid_body,
        grid=(N // BLK,),
        in_specs=[
            pl.BlockSpec((BLK, 128), lambda i: (i, 0)),
            pl.BlockSpec((BLK, 128), lambda i: (i, 0)),
            pl.BlockSpec((BLK, LANES), lambda i: (i, 0)),
            pl.BlockSpec((BLK, LANES), lambda i: (i, 0)),
            pl.BlockSpec((D_IN, H), lambda i: (0, 0)),
            pl.BlockSpec((1, H), lambda i: (0, 0)),
        ],
        out_specs=[pl.BlockSpec((BLK, 128), lambda i: (i, 0))] * 4,
        out_shape=[jax.ShapeDtypeStruct((N, 128), jnp.float32)] * 4,
    )(al, ar, dego, degi, w0, b0)


def _fin_body(a0, a1, a2, a3, degi_ref, w1, b1, wg, bg, out_ref, acc_ref):
    i = pl.program_id(0)

    @pl.when(i == 0)
    def _():
        acc_ref[:, :] = jnp.zeros_like(acc_ref)

    m = jnp.dot(a0[:, :], w1[0:128, :], preferred_element_type=jnp.float32)
    m += jnp.dot(a1[:, :], w1[128:256, :], preferred_element_type=jnp.float32)
    m += jnp.dot(a2[:, :], w1[256:384, :], preferred_element_type=jnp.float32)
    m += jnp.dot(a3[:, :], w1[384:512, :], preferred_element_type=jnp.float32)
    nd = lax.rsqrt(jnp.maximum(degi_ref[:, 0:1], 1.0))
    h2 = jnp.maximum(m * nd + b1[:, :], 0.0)
    acc_ref[:, :] += jnp.sum(h2, axis=0, keepdims=True)

    @pl.when(i == pl.num_programs(0) - 1)
    def _():
        hg = acc_ref[:, :] * (1.0 / N)
        out_ref[:, :] = (
            jnp.dot(hg, wg[:, :], preferred_element_type=jnp.float32) + bg[:, :]
        )


def _fin_call(a0, a1, a2, a3, degi, w1, b1, wg, bg):
    return pl.pallas_call(
        _fin_body,
        grid=(N // BLK,),
        in_specs=[
            pl.BlockSpec((BLK, 128), lambda i: (i, 0)),
            pl.BlockSpec((BLK, 128), lambda i: (i, 0)),
            pl.BlockSpec((BLK, 128), lambda i: (i, 0)),
            pl.BlockSpec((BLK, 128), lambda i: (i, 0)),
            pl.BlockSpec((BLK, LANES), lambda i: (i, 0)),
            pl.BlockSpec((H, H), lambda i: (0, 0)),
            pl.BlockSpec((1, H), lambda i: (0, 0)),
            pl.BlockSpec((H, D_OUT), lambda i: (0, 0)),
            pl.BlockSpec((1, D_OUT), lambda i: (0, 0)),
        ],
        out_specs=pl.BlockSpec((1, D_OUT), lambda i: (0, 0)),
        out_shape=jax.ShapeDtypeStruct((1, D_OUT), jnp.float32),
        scratch_shapes=[pltpu.VMEM((1, H), jnp.float32)],
    )(a0, a1, a2, a3, degi, w1, b1, wg, bg)


def kernel(x, edge_index, W0, b0, W1, b1, Wg, bg):
    src3 = edge_index[0].reshape(NS, NCHUNK, CH)
    dst3 = edge_index[1].reshape(NS, NCHUNK, CH)

    dego, degi = _deg_kernel(src3, dst3)

    xs_l, xs_r = _prep_call(x, dego)
    a0_l, a0_r = _agg_kernel(src3, dst3, xs_l, xs_r)

    h0, h1, h2, h3 = _mid_call(a0_l, a0_r, dego, degi, W0, b0.reshape(1, H))
    g0, g1 = _agg_kernel(src3, dst3, h0, h1)
    g2, g3 = _agg_kernel(src3, dst3, h2, h3)

    return _fin_call(
        g0, g1, g2, g3, degi, W1, b1.reshape(1, H), Wg, bg.reshape(1, D_OUT)
    )


# speedup vs baseline: 3.5663x; 3.5663x over previous
import jax
import jax.numpy as jnp
from jax import lax
from jax.experimental import pallas as pl
from jax.experimental.pallas import tpu as pltpu
from jax.experimental.pallas import tpu_sc as plsc

N = 10000
E = 160000
D_IN = 256
H = 512
D_OUT = 256

NC = 2
NS = 16
LANES = 16
W = 64

EPT = E // NS
CH = 80
NCHUNK = EPT // CH
ROWCH = N // CH

_mesh = plsc.VectorSubcoreMesh(
    core_axis_name="c", subcore_axis_name="s", num_cores=NC, num_subcores=NS
)
_sc_params = pltpu.CompilerParams(use_tc_tiling_on_sc=False)


def _zero_vmem(ref, rows, width):
    def body(r, _):
        for k in range(width // LANES):
            ref[r, pl.ds(k * LANES, LANES)] = jnp.zeros((LANES,), jnp.float32)
        return 0
    lax.fori_loop(0, rows, body, 0, unroll=False)


def _acc_chunks(s, fn):
    for k in range(ROWCH // NS):
        fn(k * NS + s)
    rem = ROWCH % NS
    if rem:
        @pl.when(s < rem)
        def _():
            fn((ROWCH // NS) * NS + s)


def _deg_body(src3, dst3, dego_hbm, degi_hbm, idx2, ones_v, zb, acc):
    c = lax.axis_index("c")
    s = lax.axis_index("s")

    def fill(r, _):
        ones_v[r, :] = jnp.full((LANES,), 1.0, jnp.float32)
        zb[r, :] = jnp.zeros((LANES,), jnp.float32)
        return 0
    lax.fori_loop(0, CH, fill, 0, unroll=False)

    _acc_chunks(s, lambda ci: pltpu.sync_copy(zb, acc.at[pl.ds(ci * CH, CH)]))
    plsc.subcore_barrier()

    @pl.when(c == 0)
    def _():
        pltpu.sync_copy(src3.at[s], idx2)

    @pl.when(c == 1)
    def _():
        pltpu.sync_copy(dst3.at[s], idx2)

    def eloop(j, _):
        pltpu.sync_copy(ones_v, acc.at[idx2.at[j]], add=True)
        return 0
    lax.fori_loop(0, NCHUNK, eloop, 0, unroll=False)
    plsc.subcore_barrier()

    @pl.when(c == 0)
    def _():
        _acc_chunks(s, lambda ci: pltpu.sync_copy(
            acc.at[pl.ds(ci * CH, CH)], dego_hbm.at[pl.ds(ci * CH, CH)]))

    @pl.when(c == 1)
    def _():
        _acc_chunks(s, lambda ci: pltpu.sync_copy(
            acc.at[pl.ds(ci * CH, CH)], degi_hbm.at[pl.ds(ci * CH, CH)]))


_deg_kernel = pl.kernel(
    _deg_body,
    out_type=[
        jax.ShapeDtypeStruct((N, LANES), jnp.float32),
        jax.ShapeDtypeStruct((N, LANES), jnp.float32),
    ],
    mesh=_mesh,
    scratch_types=[
        pltpu.VMEM((NCHUNK, CH), jnp.int32),
        pltpu.VMEM((CH, LANES), jnp.float32),
        pltpu.VMEM((CH, LANES), jnp.float32),
        pltpu.VMEM_SHARED((N, LANES), jnp.float32),
    ],
    compiler_params=_sc_params,
)


def _make_agg(spc):
    nslices = NC * spc

    def body(*refs):
        src3, dst3 = refs[0], refs[1]
        tables = refs[2:2 + nslices]
        outs = refs[2 + nslices:2 + 2 * nslices]
        sidx, didx, rows, zb, acc = refs[2 + 2 * nslices:]

        c = lax.axis_index("c")
        s = lax.axis_index("s")

        _zero_vmem(zb, CH, W)
        pltpu.sync_copy(src3.at[s], sidx)
        pltpu.sync_copy(dst3.at[s], didx)

        for t in range(nslices):
            @pl.when(c == t // spc)
            def _(t=t):
                _acc_chunks(s, lambda ci: pltpu.sync_copy(
                    zb, acc.at[pl.ds(ci * CH, CH)]))
                plsc.subcore_barrier()

                def eloop(j, _):
                    pltpu.sync_copy(tables[t].at[sidx.at[j]], rows)
                    pltpu.sync_copy(rows, acc.at[didx.at[j]], add=True)
                    return 0
                lax.fori_loop(0, NCHUNK, eloop, 0, unroll=False)
                plsc.subcore_barrier()

                _acc_chunks(s, lambda ci: pltpu.sync_copy(
                    acc.at[pl.ds(ci * CH, CH)],
                    outs[t].at[pl.ds(ci * CH, CH)]))
                plsc.subcore_barrier()

    return pl.kernel(
        body,
        out_type=[jax.ShapeDtypeStruct((N, W), jnp.float32)] * nslices,
        mesh=_mesh,
        scratch_types=[
            pltpu.VMEM((NCHUNK, CH), jnp.int32),
            pltpu.VMEM((NCHUNK, CH), jnp.int32),
            pltpu.VMEM((CH, W), jnp.float32),
            pltpu.VMEM((CH, W), jnp.float32),
            pltpu.VMEM_SHARED((N, W), jnp.float32),
        ],
        compiler_params=_sc_params,
    )


_agg4 = _make_agg(2)
_agg8 = _make_agg(4)


BLK = 1000


def _prep_body(x_ref, dego_ref, *outs):
    ns = lax.rsqrt(jnp.maximum(dego_ref[:, 0:1], 1.0))
    xs = x_ref[:, :] * ns
    for k, o in enumerate(outs):
        o[:, :] = xs[:, k * W:(k + 1) * W]


def _prep_call(x, dego):
    nsl = D_IN // W
    return pl.pallas_call(
        _prep_body,
        grid=(N // BLK,),
        in_specs=[
            pl.BlockSpec((BLK, D_IN), lambda i: (i, 0)),
            pl.BlockSpec((BLK, LANES), lambda i: (i, 0)),
        ],
        out_specs=[pl.BlockSpec((BLK, W), lambda i: (i, 0))] * nsl,
        out_shape=[jax.ShapeDtypeStruct((N, W), jnp.float32)] * nsl,
    )(x, dego)


def _mid_body(a0, a1, a2, a3, dego_ref, degi_ref, w, b, *outs):
    m = jnp.dot(a0[:, :], w[0:64, :], preferred_element_type=jnp.float32)
    m += jnp.dot(a1[:, :], w[64:128, :], preferred_element_type=jnp.float32)
    m += jnp.dot(a2[:, :], w[128:192, :], preferred_element_type=jnp.float32)
    m += jnp.dot(a3[:, :], w[192:256, :], preferred_element_type=jnp.float32)
    nd = lax.rsqrt(jnp.maximum(degi_ref[:, 0:1], 1.0))
    h = jnp.maximum(m * nd + b[:, :], 0.0)
    ns = lax.rsqrt(jnp.maximum(dego_ref[:, 0:1], 1.0))
    hs = h * ns
    for k, o in enumerate(outs):
        o[:, :] = hs[:, k * W:(k + 1) * W]


def _mid_call(a0, a1, a2, a3, dego, degi, w0, b0):
    nsl = H // W
    return pl.pallas_call(
        _mid_body,
        grid=(N // BLK,),
        in_specs=[
            pl.BlockSpec((BLK, W), lambda i: (i, 0)),
            pl.BlockSpec((BLK, W), lambda i: (i, 0)),
            pl.BlockSpec((BLK, W), lambda i: (i, 0)),
            pl.BlockSpec((BLK, W), lambda i: (i, 0)),
            pl.BlockSpec((BLK, LANES), lambda i: (i, 0)),
            pl.BlockSpec((BLK, LANES), lambda i: (i, 0)),
            pl.BlockSpec((D_IN, H), lambda i: (0, 0)),
            pl.BlockSpec((1, H), lambda i: (0, 0)),
        ],
        out_specs=[pl.BlockSpec((BLK, W), lambda i: (i, 0))] * nsl,
        out_shape=[jax.ShapeDtypeStruct((N, W), jnp.float32)] * nsl,
    )(a0, a1, a2, a3, dego, degi, w0, b0)


def _fin_body(*refs):
    gs = refs[0:8]
    degi_ref, w1, b1, wg, bg, out_ref, acc_ref = refs[8:]
    i = pl.program_id(0)

    @pl.when(i == 0)
    def _():
        acc_ref[:, :] = jnp.zeros_like(acc_ref)

    m = jnp.dot(gs[0][:, :], w1[0:64, :], preferred_element_type=jnp.float32)
    for k in range(1, 8):
        m += jnp.dot(gs[k][:, :], w1[k * 64:(k + 1) * 64, :],
                     preferred_element_type=jnp.float32)
    nd = lax.rsqrt(jnp.maximum(degi_ref[:, 0:1], 1.0))
    h2 = jnp.maximum(m * nd + b1[:, :], 0.0)
    acc_ref[:, :] += jnp.sum(h2, axis=0, keepdims=True)

    @pl.when(i == pl.num_programs(0) - 1)
    def _():
        hg = acc_ref[:, :] * (1.0 / N)
        out_ref[:, :] = (
            jnp.dot(hg, wg[:, :], preferred_element_type=jnp.float32) + bg[:, :]
        )


def _fin_call(gs, degi, w1, b1, wg, bg):
    return pl.pallas_call(
        _fin_body,
        grid=(N // BLK,),
        in_specs=(
            [pl.BlockSpec((BLK, W), lambda i: (i, 0))] * 8
            + [
                pl.BlockSpec((BLK, LANES), lambda i: (i, 0)),
                pl.BlockSpec((H, H), lambda i: (0, 0)),
                pl.BlockSpec((1, H), lambda i: (0, 0)),
                pl.BlockSpec((H, D_OUT), lambda i: (0, 0)),
                pl.BlockSpec((1, D_OUT), lambda i: (0, 0)),
            ]
        ),
        out_specs=pl.BlockSpec((1, D_OUT), lambda i: (0, 0)),
        out_shape=jax.ShapeDtypeStruct((1, D_OUT), jnp.float32),
        scratch_shapes=[pltpu.VMEM((1, H), jnp.float32)],
    )(*gs, degi, w1, b1, wg, bg)


def kernel(x, edge_index, W0, b0, W1, b1, Wg, bg):
    src3 = edge_index[0].reshape(NS, NCHUNK, CH)
    dst3 = edge_index[1].reshape(NS, NCHUNK, CH)

    dego, degi = _deg_kernel(src3, dst3)

    xs = _prep_call(x, dego)
    a = _agg4(src3, dst3, *xs)

    hs = _mid_call(*a, dego, degi, W0, b0.reshape(1, H))
    g = _agg8(src3, dst3, *hs)

    return _fin_call(g, degi, W1, b1.reshape(1, H), Wg, bg.reshape(1, D_OUT))


# speedup vs baseline: 4.2973x; 1.2050x over previous
import jax
import jax.numpy as jnp
from jax import lax
from jax.experimental import pallas as pl
from jax.experimental.pallas import tpu as pltpu
from jax.experimental.pallas import tpu_sc as plsc

N = 10000
E = 160000
D_IN = 256
H = 512
D_OUT = 256

NC = 2
NS = 16
LANES = 16
W = 64

EPT = E // NS
CH = 80
NCHUNK = EPT // CH
ROWCH = N // CH

_mesh = plsc.VectorSubcoreMesh(
    core_axis_name="c", subcore_axis_name="s", num_cores=NC, num_subcores=NS
)
_sc_params = pltpu.CompilerParams(use_tc_tiling_on_sc=False)


def _zero_vmem(ref, rows, width):
    def body(r, _):
        for k in range(width // LANES):
            ref[r, pl.ds(k * LANES, LANES)] = jnp.zeros((LANES,), jnp.float32)
        return 0
    lax.fori_loop(0, rows, body, 0, unroll=False)


def _acc_chunks(s, fn):
    for k in range(ROWCH // NS):
        fn(k * NS + s)
    rem = ROWCH % NS
    if rem:
        @pl.when(s < rem)
        def _():
            fn((ROWCH // NS) * NS + s)


def _deg_body(src3, dst3, dego_hbm, degi_hbm, idx2, ones_v, zb, acc):
    c = lax.axis_index("c")
    s = lax.axis_index("s")

    def fill(r, _):
        ones_v[r, :] = jnp.full((LANES,), 1.0, jnp.float32)
        zb[r, :] = jnp.zeros((LANES,), jnp.float32)
        return 0
    lax.fori_loop(0, CH, fill, 0, unroll=False)

    _acc_chunks(s, lambda ci: pltpu.sync_copy(zb, acc.at[pl.ds(ci * CH, CH)]))
    plsc.subcore_barrier()

    @pl.when(c == 0)
    def _():
        pltpu.sync_copy(src3.at[s], idx2)

    @pl.when(c == 1)
    def _():
        pltpu.sync_copy(dst3.at[s], idx2)

    def eloop(j, _):
        pltpu.sync_copy(ones_v, acc.at[idx2.at[j]], add=True)
        return 0
    lax.fori_loop(0, NCHUNK, eloop, 0, unroll=False)
    plsc.subcore_barrier()

    @pl.when(c == 0)
    def _():
        _acc_chunks(s, lambda ci: pltpu.sync_copy(
            acc.at[pl.ds(ci * CH, CH)], dego_hbm.at[pl.ds(ci * CH, CH)]))

    @pl.when(c == 1)
    def _():
        _acc_chunks(s, lambda ci: pltpu.sync_copy(
            acc.at[pl.ds(ci * CH, CH)], degi_hbm.at[pl.ds(ci * CH, CH)]))


_deg_kernel = pl.kernel(
    _deg_body,
    out_type=[
        jax.ShapeDtypeStruct((N, LANES), jnp.float32),
        jax.ShapeDtypeStruct((N, LANES), jnp.float32),
    ],
    mesh=_mesh,
    scratch_types=[
        pltpu.VMEM((NCHUNK, CH), jnp.int32),
        pltpu.VMEM((CH, LANES), jnp.float32),
        pltpu.VMEM((CH, LANES), jnp.float32),
        pltpu.VMEM_SHARED((N, LANES), jnp.float32),
    ],
    compiler_params=_sc_params,
)


def _make_agg(spc):
    nslices = NC * spc

    def body(*refs):
        src3, dst3 = refs[0], refs[1]
        tables = refs[2:2 + nslices]
        outs = refs[2 + nslices:2 + 2 * nslices]
        sidx, didx, rows0, rows1, zb, acc, sem0, sem1 = refs[2 + 2 * nslices:]

        c = lax.axis_index("c")
        s = lax.axis_index("s")

        _zero_vmem(zb, CH, W)
        pltpu.sync_copy(src3.at[s], sidx)
        pltpu.sync_copy(dst3.at[s], didx)

        for t in range(nslices):
            @pl.when(c == t // spc)
            def _(t=t):
                tbl = tables[t]
                _acc_chunks(s, lambda ci: pltpu.sync_copy(
                    zb, acc.at[pl.ds(ci * CH, CH)]))
                plsc.subcore_barrier()

                pltpu.async_copy(tbl.at[sidx.at[0]], rows0, sem0)

                def eloop(j, _):
                    for par, (rb, sb, nb, nsb) in enumerate(
                        ((rows0, sem0, rows1, sem1),
                         (rows1, sem1, rows0, sem0))
                    ):
                        @pl.when(lax.rem(j, 2) == par)
                        def _(rb=rb, sb=sb, nb=nb, nsb=nsb):
                            pltpu.make_async_copy(
                                tbl.at[sidx.at[j]], rb, sb).wait()

                            @pl.when(j + 1 < NCHUNK)
                            def _():
                                pltpu.async_copy(
                                    tbl.at[sidx.at[j + 1]], nb, nsb)

                            pltpu.sync_copy(rb, acc.at[didx.at[j]], add=True)
                    return 0
                lax.fori_loop(0, NCHUNK, eloop, 0, unroll=False)
                plsc.subcore_barrier()

                _acc_chunks(s, lambda ci: pltpu.sync_copy(
                    acc.at[pl.ds(ci * CH, CH)],
                    outs[t].at[pl.ds(ci * CH, CH)]))
                plsc.subcore_barrier()

    return pl.kernel(
        body,
        out_type=[jax.ShapeDtypeStruct((N, W), jnp.float32)] * nslices,
        mesh=_mesh,
        scratch_types=[
            pltpu.VMEM((NCHUNK, CH), jnp.int32),
            pltpu.VMEM((NCHUNK, CH), jnp.int32),
            pltpu.VMEM((CH, W), jnp.float32),
            pltpu.VMEM((CH, W), jnp.float32),
            pltpu.VMEM((CH, W), jnp.float32),
            pltpu.VMEM_SHARED((N, W), jnp.float32),
            pltpu.SemaphoreType.DMA,
            pltpu.SemaphoreType.DMA,
        ],
        compiler_params=_sc_params,
    )


_agg4 = _make_agg(2)
_agg8 = _make_agg(4)


BLK = 1000


def _prep_body(x_ref, dego_ref, *outs):
    ns = lax.rsqrt(jnp.maximum(dego_ref[:, 0:1], 1.0))
    xs = x_ref[:, :] * ns
    for k, o in enumerate(outs):
        o[:, :] = xs[:, k * W:(k + 1) * W]


def _prep_call(x, dego):
    nsl = D_IN // W
    return pl.pallas_call(
        _prep_body,
        grid=(N // BLK,),
        in_specs=[
            pl.BlockSpec((BLK, D_IN), lambda i: (i, 0)),
            pl.BlockSpec((BLK, LANES), lambda i: (i, 0)),
        ],
        out_specs=[pl.BlockSpec((BLK, W), lambda i: (i, 0))] * nsl,
        out_shape=[jax.ShapeDtypeStruct((N, W), jnp.float32)] * nsl,
    )(x, dego)


def _mid_body(a0, a1, a2, a3, dego_ref, degi_ref, w, b, *outs):
    m = jnp.dot(a0[:, :], w[0:64, :], preferred_element_type=jnp.float32)
    m += jnp.dot(a1[:, :], w[64:128, :], preferred_element_type=jnp.float32)
    m += jnp.dot(a2[:, :], w[128:192, :], preferred_element_type=jnp.float32)
    m += jnp.dot(a3[:, :], w[192:256, :], preferred_element_type=jnp.float32)
    nd = lax.rsqrt(jnp.maximum(degi_ref[:, 0:1], 1.0))
    h = jnp.maximum(m * nd + b[:, :], 0.0)
    ns = lax.rsqrt(jnp.maximum(dego_ref[:, 0:1], 1.0))
    hs = h * ns
    for k, o in enumerate(outs):
        o[:, :] = hs[:, k * W:(k + 1) * W]


def _mid_call(a0, a1, a2, a3, dego, degi, w0, b0):
    nsl = H // W
    return pl.pallas_call(
        _mid_body,
        grid=(N // BLK,),
        in_specs=[
            pl.BlockSpec((BLK, W), lambda i: (i, 0)),
            pl.BlockSpec((BLK, W), lambda i: (i, 0)),
            pl.BlockSpec((BLK, W), lambda i: (i, 0)),
            pl.BlockSpec((BLK, W), lambda i: (i, 0)),
            pl.BlockSpec((BLK, LANES), lambda i: (i, 0)),
            pl.BlockSpec((BLK, LANES), lambda i: (i, 0)),
            pl.BlockSpec((D_IN, H), lambda i: (0, 0)),
            pl.BlockSpec((1, H), lambda i: (0, 0)),
        ],
        out_specs=[pl.BlockSpec((BLK, W), lambda i: (i, 0))] * nsl,
        out_shape=[jax.ShapeDtypeStruct((N, W), jnp.float32)] * nsl,
    )(a0, a1, a2, a3, dego, degi, w0, b0)


def _fin_body(*refs):
    gs = refs[0:8]
    degi_ref, w1, b1, wg, bg, out_ref, acc_ref = refs[8:]
    i = pl.program_id(0)

    @pl.when(i == 0)
    def _():
        acc_ref[:, :] = jnp.zeros_like(acc_ref)

    m = jnp.dot(gs[0][:, :], w1[0:64, :], preferred_element_type=jnp.float32)
    for k in range(1, 8):
        m += jnp.dot(gs[k][:, :], w1[k * 64:(k + 1) * 64, :],
                     preferred_element_type=jnp.float32)
    nd = lax.rsqrt(jnp.maximum(degi_ref[:, 0:1], 1.0))
    h2 = jnp.maximum(m * nd + b1[:, :], 0.0)
    acc_ref[:, :] += jnp.sum(h2, axis=0, keepdims=True)

    @pl.when(i == pl.num_programs(0) - 1)
    def _():
        hg = acc_ref[:, :] * (1.0 / N)
        out_ref[:, :] = (
            jnp.dot(hg, wg[:, :], preferred_element_type=jnp.float32) + bg[:, :]
        )


def _fin_call(gs, degi, w1, b1, wg, bg):
    return pl.pallas_call(
        _fin_body,
        grid=(N // BLK,),
        in_specs=(
            [pl.BlockSpec((BLK, W), lambda i: (i, 0))] * 8
            + [
                pl.BlockSpec((BLK, LANES), lambda i: (i, 0)),
                pl.BlockSpec((H, H), lambda i: (0, 0)),
                pl.BlockSpec((1, H), lambda i: (0, 0)),
                pl.BlockSpec((H, D_OUT), lambda i: (0, 0)),
                pl.BlockSpec((1, D_OUT), lambda i: (0, 0)),
            ]
        ),
        out_specs=pl.BlockSpec((1, D_OUT), lambda i: (0, 0)),
        out_shape=jax.ShapeDtypeStruct((1, D_OUT), jnp.float32),
        scratch_shapes=[pltpu.VMEM((1, H), jnp.float32)],
    )(*gs, degi, w1, b1, wg, bg)


def kernel(x, edge_index, W0, b0, W1, b1, Wg, bg):
    src3 = edge_index[0].reshape(NS, NCHUNK, CH)
    dst3 = edge_index[1].reshape(NS, NCHUNK, CH)

    dego, degi = _deg_kernel(src3, dst3)

    xs = _prep_call(x, dego)
    a = _agg4(src3, dst3, *xs)

    hs = _mid_call(*a, dego, degi, W0, b0.reshape(1, H))
    g = _agg8(src3, dst3, *hs)

    return _fin_call(g, degi, W1, b1.reshape(1, H), Wg, bg.reshape(1, D_OUT))
